# Initial kernel scaffold; baseline (speedup 1.0000x reference)
import jax
import jax.numpy as jnp
from jax.experimental import pallas as pl


def kernel(x, edge_index, pred_edges, Wl0, bl0, Wr0, g0, be0, Wl1, bl1, Wr1, g1, be1, Wl2, bl2, Wr2, g2, be2, W1, b1, W2, b2, W3, b3):
    raise NotImplementedError("write your pallas kernel here")


# speedup vs baseline: 2.1359x; 2.1359x over previous
import functools

import jax
import jax.numpy as jnp
from jax import lax
from jax.experimental import pallas as pl
from jax.experimental.pallas import tpu as pltpu
from jax.experimental.pallas import tpu_sc as plsc

N = 10000
D = 256
DH = 128
H = 128
E = 160000
PE = 160000
EPAD = 163840
ROWS = EPAD // 128
NACC = 10240
PAD_DST = 10008
NC = 2
NS = 16
SLAB = NACC // NS
BN = 1000
BE = 2000

_f32 = jnp.float32
_i32 = jnp.int32

@functools.cache
def _mesh():
    return plsc.VectorSubcoreMesh(core_axis_name="c", subcore_axis_name="s",
                                  num_cores=NC, num_subcores=NS)


def _zero_rows(buf, nrows, width):
    zv = jnp.zeros((16,), _f32)

    def body(i, _):
        for j in range(width // 16):
            buf[i, pl.ds(j * 16, 16)] = zv
        return 0

    lax.fori_loop(0, nrows, body, 0)


@functools.cache
def _build_sc_count():
    return functools.partial(
        pl.kernel,
        out_type=jax.ShapeDtypeStruct((NC * NACC, 128), _f32),
        mesh=_mesh(),
        scratch_types=[
            pltpu.VMEM((40, 128), _i32),
            pltpu.VMEM((128, 128), _f32),
            pltpu.VMEM((64, 128), _f32),
            pltpu.VMEM_SHARED((NACC, 128), _f32),
        ],
    )(_sc_count_body)


def _sc_count(dstp):
    return _build_sc_count()(dstp)


def _sc_count_body(dstp, cnt_out, ibuf, ones_b, zbuf, acc):
    c = lax.axis_index("c")
    s = lax.axis_index("s")
    w = s * NC + c

    ov = jnp.ones((16,), _f32)

    def fill_ones(i, _):
        for j in range(8):
            ones_b[i, pl.ds(j * 16, 16)] = ov
        return 0

    lax.fori_loop(0, 128, fill_ones, 0)
    _zero_rows(zbuf, 64, 128)

    def zcopy(k, _):
        pltpu.sync_copy(zbuf, acc.at[pl.ds(s * SLAB + k * 64, 64)])
        return 0

    lax.fori_loop(0, SLAB // 64, zcopy, 0)
    plsc.subcore_barrier()

    pltpu.sync_copy(dstp.at[pl.ds(w * 40, 40)], ibuf)

    def body(j, _):
        pltpu.sync_copy(ones_b, acc.at[ibuf.at[j]], add=True)
        return 0

    lax.fori_loop(0, 40, body, 0)
    plsc.subcore_barrier()

    def wb(k, _):
        pltpu.sync_copy(acc.at[pl.ds(s * SLAB + k * 64, 64)], zbuf)
        pltpu.sync_copy(zbuf, cnt_out.at[pl.ds(c * NACC + s * SLAB + k * 64, 64)])
        return 0

    lax.fori_loop(0, SLAB // 64, wb, 0)


@functools.cache
def _build_sc_agg():
    return functools.partial(
        pl.kernel,
        out_type=jax.ShapeDtypeStruct((NC * NACC, DH), _f32),
        mesh=_mesh(),
        scratch_types=[
            pltpu.VMEM((2, 128), _i32),
            pltpu.VMEM((2, 128), _i32),
            pltpu.VMEM((256, DH), _f32),
            pltpu.VMEM((16, DH), _f32),
            pltpu.VMEM_SHARED((NACC, DH), _f32),
            pltpu.SemaphoreType.DMA,
        ],
    )(_sc_agg_body)


def _sc_agg(m01, srcp, dstp):
    return _build_sc_agg()(m01, srcp, dstp)


def _sc_agg_body(m01, srcp, dstp, agg_out, isrc, idst, gbuf, zbuf, acc, sem):
    c = lax.axis_index("c")
    s = lax.axis_index("s")

    _zero_rows(zbuf, 16, DH)

    def zcopy(k, _):
        pltpu.sync_copy(zbuf, acc.at[pl.ds(s * SLAB + k * 16, 16)])
        return 0

    lax.fori_loop(0, SLAB // 16, zcopy, 0)
    plsc.subcore_barrier()

    off = (c * N).astype(_i32)

    def body(it, _):
        rbase = s * 80 + it * 2
        pltpu.sync_copy(srcp.at[pl.ds(rbase, 2)], isrc)
        pltpu.sync_copy(dstp.at[pl.ds(rbase, 2)], idst)
        for rr in range(2):
            for j in range(8):
                isrc[rr, pl.ds(j * 16, 16)] = isrc[rr, pl.ds(j * 16, 16)] + off
        descs = [
            pltpu.async_copy(m01.at[isrc.at[rr]],
                             gbuf.at[pl.ds(rr * 128, 128)], sem)
            for rr in range(2)
        ]
        for d in descs:
            d.wait()
        for rr in range(2):
            pltpu.sync_copy(gbuf.at[pl.ds(rr * 128, 128)],
                            acc.at[idst.at[rr]], add=True)
        return 0

    lax.fori_loop(0, 40, body, 0)
    plsc.subcore_barrier()

    def wb(k, _):
        pltpu.sync_copy(acc.at[pl.ds(s * SLAB + k * 128, 128)],
                        gbuf.at[pl.ds(0, 128)])
        pltpu.sync_copy(gbuf.at[pl.ds(0, 128)],
                        agg_out.at[pl.ds(c * NACC + s * SLAB + k * 128, 128)])
        return 0

    lax.fori_loop(0, SLAB // 128, wb, 0)


@functools.cache
def _build_sc_pred():
    return functools.partial(
        pl.kernel,
        out_type=[
            jax.ShapeDtypeStruct((EPAD, DH), _f32),
            jax.ShapeDtypeStruct((EPAD, DH), _f32),
        ],
        mesh=_mesh(),
        scratch_types=[
            pltpu.VMEM((2, 128), _i32),
            pltpu.VMEM((2, 128), _i32),
            pltpu.VMEM((256, DH), _f32),
            pltpu.VMEM((256, DH), _f32),
            pltpu.SemaphoreType.DMA,
        ],
    )(_sc_pred_body)


def _sc_pred(a_h, b_h, psrcp, ptgtp):
    return _build_sc_pred()(a_h, b_h, psrcp, ptgtp)


def _sc_pred_body(a_h, b_h, psrcp, ptgtp, ca_out, cb_out, isrc, itgt, bufa,
                  bufb, sem):
    c = lax.axis_index("c")
    s = lax.axis_index("s")
    w = s * NC + c

    def body(it, _):
        rbase = w * 40 + it * 2
        pltpu.sync_copy(psrcp.at[pl.ds(rbase, 2)], isrc)
        pltpu.sync_copy(ptgtp.at[pl.ds(rbase, 2)], itgt)
        descs = []
        for rr in range(2):
            descs.append(pltpu.async_copy(
                a_h.at[isrc.at[rr]], bufa.at[pl.ds(rr * 128, 128)], sem))
            descs.append(pltpu.async_copy(
                b_h.at[itgt.at[rr]], bufb.at[pl.ds(rr * 128, 128)], sem))
        for d in descs:
            d.wait()
        ebase = w * 5120 + it * 256
        pltpu.sync_copy(bufa, ca_out.at[pl.ds(ebase, 256)])
        pltpu.sync_copy(bufb, cb_out.at[pl.ds(ebase, 256)])
        return 0

    lax.fori_loop(0, 20, body, 0)


def _dot_t(x, w):
    return lax.dot_general(x, w, (((1,), (1,)), ((), ())),
                           preferred_element_type=_f32)


def _tc_i0_body(x_ref, wl_ref, wr_ref, m01_ref, r_ref):
    xb = x_ref[...]
    m = _dot_t(xb, wl_ref[...])
    m01_ref[0] = m[:, :DH]
    m01_ref[1] = m[:, DH:]
    r_ref[...] = _dot_t(xb, wr_ref[...])


def _tc_i0(x, wl, wr):
    grid = N // BN
    return pl.pallas_call(
        _tc_i0_body,
        grid=(grid,),
        in_specs=[
            pl.BlockSpec((BN, D), lambda i: (i, 0)),
            pl.BlockSpec((D, D), lambda i: (0, 0)),
            pl.BlockSpec((D, D), lambda i: (0, 0)),
        ],
        out_specs=[
            pl.BlockSpec((2, BN, DH), lambda i: (0, i, 0)),
            pl.BlockSpec((BN, D), lambda i: (i, 0)),
        ],
        out_shape=[
            jax.ShapeDtypeStruct((2, N, DH), _f32),
            jax.ShapeDtypeStruct((N, D), _f32),
        ],
    )(x, wl, wr)


def _tc_stats_body(agga_ref, aggb_ref, cnta_ref, cntb_ref, r_ref, bl_ref,
                   pre_ref, s1_ref, s2_ref):
    agg = jnp.concatenate([agga_ref[0], aggb_ref[0]], axis=1)
    cnt = cnta_ref[0][:, :1] + cntb_ref[0][:, :1]
    inv = 1.0 / jnp.maximum(cnt, 1.0)
    pre = agg * inv + bl_ref[...] + r_ref[...]
    pre_ref[...] = pre

    @pl.when(pl.program_id(0) == 0)
    def _():
        s1_ref[...] = jnp.zeros((1, D), _f32)
        s2_ref[...] = jnp.zeros((1, D), _f32)

    s1_ref[...] += jnp.sum(pre, axis=0, keepdims=True)
    s2_ref[...] += jnp.sum(pre * pre, axis=0, keepdims=True)


def _tc_stats(agg2, cnt2, r, bl):
    grid = N // BN
    return pl.pallas_call(
        _tc_stats_body,
        grid=(grid,),
        in_specs=[
            pl.BlockSpec((1, BN, DH), lambda i: (0, i, 0)),
            pl.BlockSpec((1, BN, DH), lambda i: (1, i, 0)),
            pl.BlockSpec((1, BN, 128), lambda i: (0, i, 0)),
            pl.BlockSpec((1, BN, 128), lambda i: (1, i, 0)),
            pl.BlockSpec((BN, D), lambda i: (i, 0)),
            pl.BlockSpec((1, D), lambda i: (0, 0)),
        ],
        out_specs=[
            pl.BlockSpec((BN, D), lambda i: (i, 0)),
            pl.BlockSpec((1, D), lambda i: (0, 0)),
            pl.BlockSpec((1, D), lambda i: (0, 0)),
        ],
        out_shape=[
            jax.ShapeDtypeStruct((N, D), _f32),
            jax.ShapeDtypeStruct((1, D), _f32),
            jax.ShapeDtypeStruct((1, D), _f32),
        ],
    )(agg2, agg2, cnt2, cnt2, r, bl)


def _bn_relu(pre, s1, s2, g, be):
    mean = s1 * (1.0 / N)
    var = s2 * (1.0 / N) - mean * mean
    scale = g / jnp.sqrt(var + 1e-5)
    shift = be - mean * scale
    return jnp.maximum(pre * scale + shift, 0.0)


def _tc_fin01_body(pre_ref, h_ref, s1_ref, s2_ref, g_ref, be_ref,
                   wln_ref, wrn_ref, h_out_ref, m01_ref, r_ref):
    hn = _bn_relu(pre_ref[...], s1_ref[...], s2_ref[...],
                  g_ref[...], be_ref[...]) + h_ref[...]
    h_out_ref[...] = hn
    m = _dot_t(hn, wln_ref[...])
    m01_ref[0] = m[:, :DH]
    m01_ref[1] = m[:, DH:]
    r_ref[...] = _dot_t(hn, wrn_ref[...])


def _tc_fin01(pre, h, s1, s2, g, be, wln, wrn):
    grid = N // BN
    vec = pl.BlockSpec((1, D), lambda i: (0, 0))
    mat = pl.BlockSpec((D, D), lambda i: (0, 0))
    blk = pl.BlockSpec((BN, D), lambda i: (i, 0))
    return pl.pallas_call(
        _tc_fin01_body,
        grid=(grid,),
        in_specs=[blk, blk, vec, vec, vec, vec, mat, mat],
        out_specs=[
            blk,
            pl.BlockSpec((2, BN, DH), lambda i: (0, i, 0)),
            blk,
        ],
        out_shape=[
            jax.ShapeDtypeStruct((N, D), _f32),
            jax.ShapeDtypeStruct((2, N, DH), _f32),
            jax.ShapeDtypeStruct((N, D), _f32),
        ],
    )(pre, h, s1, s2, g, be, wln, wrn)


def _tc_fin2_body(pre_ref, s1_ref, s2_ref, g_ref, be_ref,
                  w1a_ref, w1b_ref, a_ref, b_ref):
    h3 = _bn_relu(pre_ref[...], s1_ref[...], s2_ref[...],
                  g_ref[...], be_ref[...])
    a_ref[...] = _dot_t(h3, w1a_ref[...])
    b_ref[...] = _dot_t(h3, w1b_ref[...])


def _tc_fin2(pre, s1, s2, g, be, w1a, w1b):
    grid = N // BN
    vec = pl.BlockSpec((1, D), lambda i: (0, 0))
    return pl.pallas_call(
        _tc_fin2_body,
        grid=(grid,),
        in_specs=[
            pl.BlockSpec((BN, D), lambda i: (i, 0)),
            vec, vec, vec, vec,
            pl.BlockSpec((H, D), lambda i: (0, 0)),
            pl.BlockSpec((H, D), lambda i: (0, 0)),
        ],
        out_specs=[
            pl.BlockSpec((BN, H), lambda i: (i, 0)),
            pl.BlockSpec((BN, H), lambda i: (i, 0)),
        ],
        out_shape=[
            jax.ShapeDtypeStruct((N, H), _f32),
            jax.ShapeDtypeStruct((N, H), _f32),
        ],
    )(pre, s1, s2, g, be, w1a, w1b)


def _tc_edge_body(ca_ref, cb_ref, b1_ref, w2_ref, b2_ref, w3_ref, b3_ref,
                  out_ref):
    z = jnp.maximum(ca_ref[...] + cb_ref[...] + b1_ref[...], 0.0)
    z2 = jnp.maximum(_dot_t(z, w2_ref[...]) + b2_ref[...], 0.0)
    out_ref[...] = jnp.sum(z2 * w3_ref[...], axis=1, keepdims=True) + b3_ref[...]


def _tc_edge(ca, cb, b1, w2, b2, w3, b3):
    grid = PE // BE
    return pl.pallas_call(
        _tc_edge_body,
        grid=(grid,),
        in_specs=[
            pl.BlockSpec((BE, DH), lambda i: (i, 0)),
            pl.BlockSpec((BE, DH), lambda i: (i, 0)),
            pl.BlockSpec((1, H), lambda i: (0, 0)),
            pl.BlockSpec((H // 2, H), lambda i: (0, 0)),
            pl.BlockSpec((1, H // 2), lambda i: (0, 0)),
            pl.BlockSpec((1, H // 2), lambda i: (0, 0)),
            pl.BlockSpec((1, 1), lambda i: (0, 0)),
        ],
        out_specs=pl.BlockSpec((BE, 1), lambda i: (i, 0)),
        out_shape=jax.ShapeDtypeStruct((PE, 1), _f32),
    )(ca, cb, b1, w2, b2, w3, b3)


def kernel(x, edge_index, pred_edges, Wl0, bl0, Wr0, g0, be0, Wl1, bl1, Wr1,
           g1, be1, Wl2, bl2, Wr2, g2, be2, W1, b1, W2, b2, W3, b3):
    pad = EPAD - E
    srcp = jnp.concatenate(
        [edge_index[0], jnp.zeros((pad,), _i32)]).reshape(ROWS, 128)
    dstp = jnp.concatenate(
        [edge_index[1], jnp.full((pad,), PAD_DST, _i32)]).reshape(ROWS, 128)
    psp = jnp.concatenate(
        [pred_edges[0], jnp.zeros((pad,), _i32)]).reshape(ROWS, 128)
    ptp = jnp.concatenate(
        [pred_edges[1], jnp.zeros((pad,), _i32)]).reshape(ROWS, 128)

    cnt2 = _sc_count(dstp).reshape(2, NACC, 128)

    r1d = lambda v: v.reshape(1, -1)

    m01, r = _tc_i0(x, Wl0, Wr0)
    agg2 = _sc_agg(m01.reshape(2 * N, DH), srcp, dstp).reshape(2, NACC, DH)
    pre, s1, s2 = _tc_stats(agg2, cnt2, r, r1d(bl0))
    h, m01, r = _tc_fin01(pre, x, s1, s2, r1d(g0), r1d(be0), Wl1, Wr1)

    agg2 = _sc_agg(m01.reshape(2 * N, DH), srcp, dstp).reshape(2, NACC, DH)
    pre, s1, s2 = _tc_stats(agg2, cnt2, r, r1d(bl1))
    h, m01, r = _tc_fin01(pre, h, s1, s2, r1d(g1), r1d(be1), Wl2, Wr2)

    agg2 = _sc_agg(m01.reshape(2 * N, DH), srcp, dstp).reshape(2, NACC, DH)
    pre, s1, s2 = _tc_stats(agg2, cnt2, r, r1d(bl2))
    a_nodes, b_nodes = _tc_fin2(pre, s1, s2, r1d(g2), r1d(be2),
                                W1[:, :D], W1[:, D:])

    ca, cb = _sc_pred(a_nodes, b_nodes, psp, ptp)
    z = _tc_edge(ca[:PE], cb[:PE], r1d(b1), W2, r1d(b2), W3, b3.reshape(1, 1))
    return z.reshape(-1)


# speedup vs baseline: 2.4981x; 1.1696x over previous
import functools

import jax
import jax.numpy as jnp
from jax import lax
from jax.experimental import pallas as pl
from jax.experimental.pallas import tpu as pltpu
from jax.experimental.pallas import tpu_sc as plsc

N = 10000
D = 256
DH = 128
H = 128
E = 160000
PE = 160000
EPAD = 163840
ROWS = EPAD // 128
NACC = 10240
PAD_DST = 10008
NC = 2
NS = 16
SLAB = NACC // NS
BN = 1000
BE = 2000

_f32 = jnp.float32
_i32 = jnp.int32

@functools.cache
def _mesh():
    return plsc.VectorSubcoreMesh(core_axis_name="c", subcore_axis_name="s",
                                  num_cores=NC, num_subcores=NS)


def _zero_rows(buf, nrows, width):
    zv = jnp.zeros((16,), _f32)

    def body(i, _):
        for j in range(width // 16):
            buf[i, pl.ds(j * 16, 16)] = zv
        return 0

    lax.fori_loop(0, nrows, body, 0)


@functools.cache
def _build_sc_count():
    return functools.partial(
        pl.kernel,
        out_type=jax.ShapeDtypeStruct((NC * NACC, 128), _f32),
        mesh=_mesh(),
        scratch_types=[
            pltpu.VMEM((40, 128), _i32),
            pltpu.VMEM((128, 128), _f32),
            pltpu.VMEM((64, 128), _f32),
            pltpu.VMEM_SHARED((NACC, 128), _f32),
        ],
    )(_sc_count_body)


def _sc_count(dstp):
    return _build_sc_count()(dstp)


def _sc_count_body(dstp, cnt_out, ibuf, ones_b, zbuf, acc):
    c = lax.axis_index("c")
    s = lax.axis_index("s")
    w = s * NC + c

    ov = jnp.ones((16,), _f32)

    def fill_ones(i, _):
        for j in range(8):
            ones_b[i, pl.ds(j * 16, 16)] = ov
        return 0

    lax.fori_loop(0, 128, fill_ones, 0)
    _zero_rows(zbuf, 64, 128)

    def zcopy(k, _):
        pltpu.sync_copy(zbuf, acc.at[pl.ds(s * SLAB + k * 64, 64)])
        return 0

    lax.fori_loop(0, SLAB // 64, zcopy, 0)
    plsc.subcore_barrier()

    pltpu.sync_copy(dstp.at[pl.ds(w * 40, 40)], ibuf)

    def body(j, _):
        pltpu.sync_copy(ones_b, acc.at[ibuf.at[j]], add=True)
        return 0

    lax.fori_loop(0, 40, body, 0)
    plsc.subcore_barrier()

    def wb(k, _):
        pltpu.sync_copy(acc.at[pl.ds(s * SLAB + k * 64, 64)], zbuf)
        pltpu.sync_copy(zbuf, cnt_out.at[pl.ds(c * NACC + s * SLAB + k * 64, 64)])
        return 0

    lax.fori_loop(0, SLAB // 64, wb, 0)


@functools.cache
def _build_sc_agg():
    return functools.partial(
        pl.kernel,
        out_type=jax.ShapeDtypeStruct((NC * NACC, DH), _f32),
        mesh=_mesh(),
        scratch_types=[
            pltpu.VMEM((8, 128), _i32),
            pltpu.VMEM((8, 128), _i32),
            pltpu.VMEM((8, 128), _i32),
            pltpu.VMEM((8, 128), _i32),
            pltpu.VMEM((128, DH), _f32),
            pltpu.VMEM((128, DH), _f32),
            pltpu.VMEM((16, DH), _f32),
            pltpu.VMEM_SHARED((NACC, DH), _f32),
            pltpu.SemaphoreType.DMA,
        ],
    )(_sc_agg_body)


def _sc_agg(m01, srcp, dstp):
    return _build_sc_agg()(m01, srcp, dstp)


def _sc_agg_body(m01, srcp, dstp, agg_out, isrc0, idst0, isrc1, idst1,
                 gbuf0, gbuf1, zbuf, acc, sem):
    c = lax.axis_index("c")
    s = lax.axis_index("s")
    gbufs = (gbuf0, gbuf1)

    _zero_rows(zbuf, 16, DH)

    def zcopy(k, _):
        pltpu.sync_copy(zbuf, acc.at[pl.ds(s * SLAB + k * 16, 16)])
        return 0

    lax.fori_loop(0, SLAB // 16, zcopy, 0)
    plsc.subcore_barrier()

    off = (c * N).astype(_i32)

    def load_win(wi, wd, w):
        rb = s * 80 + w * 8
        pltpu.sync_copy(srcp.at[pl.ds(rb, 8)], wi)
        pltpu.sync_copy(dstp.at[pl.ds(rb, 8)], wd)
        for r in range(8):
            for j in range(8):
                wi[r, pl.ds(j * 16, 16)] = wi[r, pl.ds(j * 16, 16)] + off

    def fire(wi, r, buf):
        pltpu.async_copy(m01.at[wi.at[r]], buf, sem)

    def wait_g(buf):
        pltpu.make_async_copy(m01.at[pl.ds(0, 128)], buf, sem).wait()

    def do_pass(wi, wd, nwi, last_pred):
        for r in range(8):
            wait_g(gbufs[r % 2])
            if r < 7:
                fire(wi, r + 1, gbufs[(r + 1) % 2])
            elif last_pred is None:
                fire(nwi, 0, gbuf0)
            else:
                @pl.when(last_pred)
                def _():
                    fire(nwi, 0, gbuf0)
            pltpu.sync_copy(gbufs[r % 2], acc.at[wd.at[r]], add=True)

    load_win(isrc0, idst0, 0)
    fire(isrc0, 0, gbuf0)

    def body(k, _):
        load_win(isrc1, idst1, 2 * k + 1)
        do_pass(isrc0, idst0, isrc1, None)

        @pl.when(k < 4)
        def _():
            load_win(isrc0, idst0, 2 * k + 2)

        do_pass(isrc1, idst1, isrc0, k < 4)
        return 0

    lax.fori_loop(0, 5, body, 0)
    plsc.subcore_barrier()

    def wb(k, _):
        pltpu.sync_copy(acc.at[pl.ds(s * SLAB + k * 128, 128)], gbuf0)
        pltpu.sync_copy(gbuf0,
                        agg_out.at[pl.ds(c * NACC + s * SLAB + k * 128, 128)])
        return 0

    lax.fori_loop(0, SLAB // 128, wb, 0)


@functools.cache
def _build_sc_pred():
    return functools.partial(
        pl.kernel,
        out_type=jax.ShapeDtypeStruct((EPAD, DH), _f32),
        mesh=_mesh(),
        scratch_types=[
            pltpu.VMEM((40, 128), _i32),
            pltpu.VMEM((40, 128), _i32),
            pltpu.VMEM((128, DH), _f32),
            pltpu.VMEM((128, DH), _f32),
            pltpu.VMEM((128, DH), _f32),
            pltpu.VMEM((128, DH), _f32),
            pltpu.SemaphoreType.DMA,
            pltpu.SemaphoreType.DMA,
            pltpu.SemaphoreType.DMA,
        ],
    )(_sc_pred_body)


def _sc_pred(a_h, b_h, psrcp, ptgtp):
    return _build_sc_pred()(a_h, b_h, psrcp, ptgtp)


def _sc_pred_body(a_h, b_h, psrcp, ptgtp, c_out, isrc, itgt,
                  bufa0, bufb0, bufa1, bufb1, semg, semw0, semw1):
    c = lax.axis_index("c")
    s = lax.axis_index("s")
    w = s * NC + c
    wbase = w * 5120

    pltpu.sync_copy(psrcp.at[pl.ds(w * 40, 40)], isrc)
    pltpu.sync_copy(ptgtp.at[pl.ds(w * 40, 40)], itgt)

    def fire_g(g, ba, bb):
        pltpu.async_copy(a_h.at[isrc.at[g]], ba, semg)
        pltpu.async_copy(b_h.at[itgt.at[g]], bb, semg)

    def wait_g(ba, bb):
        pltpu.make_async_copy(a_h.at[pl.ds(0, 128)], ba, semg).wait()
        pltpu.make_async_copy(a_h.at[pl.ds(0, 128)], bb, semg).wait()

    def vadd(ba, bb):
        def vrow(r, _):
            for j in range(8):
                ba[r, pl.ds(j * 16, 16)] = (ba[r, pl.ds(j * 16, 16)] +
                                            bb[r, pl.ds(j * 16, 16)])
            return 0

        lax.fori_loop(0, 128, vrow, 0)

    def fire_w(g, ba, semw):
        pltpu.async_copy(ba, c_out.at[pl.ds(wbase + g * 128, 128)], semw)

    def wait_w(ba, semw):
        pltpu.make_async_copy(ba, c_out.at[pl.ds(0, 128)], semw).wait()

    fire_g(0, bufa0, bufb0)

    def body(k, _):
        g0 = 2 * k
        g1 = 2 * k + 1
        wait_g(bufa0, bufb0)

        @pl.when(k > 0)
        def _():
            wait_w(bufa1, semw1)

        fire_g(g1, bufa1, bufb1)
        vadd(bufa0, bufb0)
        fire_w(g0, bufa0, semw0)
        wait_g(bufa1, bufb1)

        @pl.when(k < 19)
        def _():
            wait_w(bufa0, semw0)
            fire_g(g1 + 1, bufa0, bufb0)

        vadd(bufa1, bufb1)
        fire_w(g1, bufa1, semw1)
        return 0

    lax.fori_loop(0, 20, body, 0)
    wait_w(bufa0, semw0)
    wait_w(bufa1, semw1)


def _dot_t(x, w):
    return lax.dot_general(x, w, (((1,), (1,)), ((), ())),
                           preferred_element_type=_f32)


def _tc_i0_body(x_ref, wl_ref, wr_ref, m01_ref, r_ref):
    xb = x_ref[...]
    m = _dot_t(xb, wl_ref[...])
    m01_ref[0] = m[:, :DH]
    m01_ref[1] = m[:, DH:]
    r_ref[...] = _dot_t(xb, wr_ref[...])


def _tc_i0(x, wl, wr):
    grid = N // BN
    return pl.pallas_call(
        _tc_i0_body,
        grid=(grid,),
        in_specs=[
            pl.BlockSpec((BN, D), lambda i: (i, 0)),
            pl.BlockSpec((D, D), lambda i: (0, 0)),
            pl.BlockSpec((D, D), lambda i: (0, 0)),
        ],
        out_specs=[
            pl.BlockSpec((2, BN, DH), lambda i: (0, i, 0)),
            pl.BlockSpec((BN, D), lambda i: (i, 0)),
        ],
        out_shape=[
            jax.ShapeDtypeStruct((2, N, DH), _f32),
            jax.ShapeDtypeStruct((N, D), _f32),
        ],
    )(x, wl, wr)


def _tc_stats_body(agga_ref, aggb_ref, cnta_ref, cntb_ref, r_ref, bl_ref,
                   pre_ref, s1_ref, s2_ref):
    agg = jnp.concatenate([agga_ref[0], aggb_ref[0]], axis=1)
    cnt = cnta_ref[0][:, :1] + cntb_ref[0][:, :1]
    inv = 1.0 / jnp.maximum(cnt, 1.0)
    pre = agg * inv + bl_ref[...] + r_ref[...]
    pre_ref[...] = pre

    @pl.when(pl.program_id(0) == 0)
    def _():
        s1_ref[...] = jnp.zeros((1, D), _f32)
        s2_ref[...] = jnp.zeros((1, D), _f32)

    s1_ref[...] += jnp.sum(pre, axis=0, keepdims=True)
    s2_ref[...] += jnp.sum(pre * pre, axis=0, keepdims=True)


def _tc_stats(agg2, cnt2, r, bl):
    grid = N // BN
    return pl.pallas_call(
        _tc_stats_body,
        grid=(grid,),
        in_specs=[
            pl.BlockSpec((1, BN, DH), lambda i: (0, i, 0)),
            pl.BlockSpec((1, BN, DH), lambda i: (1, i, 0)),
            pl.BlockSpec((1, BN, 128), lambda i: (0, i, 0)),
            pl.BlockSpec((1, BN, 128), lambda i: (1, i, 0)),
            pl.BlockSpec((BN, D), lambda i: (i, 0)),
            pl.BlockSpec((1, D), lambda i: (0, 0)),
        ],
        out_specs=[
            pl.BlockSpec((BN, D), lambda i: (i, 0)),
            pl.BlockSpec((1, D), lambda i: (0, 0)),
            pl.BlockSpec((1, D), lambda i: (0, 0)),
        ],
        out_shape=[
            jax.ShapeDtypeStruct((N, D), _f32),
            jax.ShapeDtypeStruct((1, D), _f32),
            jax.ShapeDtypeStruct((1, D), _f32),
        ],
    )(agg2, agg2, cnt2, cnt2, r, bl)


def _bn_relu(pre, s1, s2, g, be):
    mean = s1 * (1.0 / N)
    var = s2 * (1.0 / N) - mean * mean
    scale = g / jnp.sqrt(var + 1e-5)
    shift = be - mean * scale
    return jnp.maximum(pre * scale + shift, 0.0)


def _tc_fin01_body(pre_ref, h_ref, s1_ref, s2_ref, g_ref, be_ref,
                   wln_ref, wrn_ref, h_out_ref, m01_ref, r_ref):
    hn = _bn_relu(pre_ref[...], s1_ref[...], s2_ref[...],
                  g_ref[...], be_ref[...]) + h_ref[...]
    h_out_ref[...] = hn
    m = _dot_t(hn, wln_ref[...])
    m01_ref[0] = m[:, :DH]
    m01_ref[1] = m[:, DH:]
    r_ref[...] = _dot_t(hn, wrn_ref[...])


def _tc_fin01(pre, h, s1, s2, g, be, wln, wrn):
    grid = N // BN
    vec = pl.BlockSpec((1, D), lambda i: (0, 0))
    mat = pl.BlockSpec((D, D), lambda i: (0, 0))
    blk = pl.BlockSpec((BN, D), lambda i: (i, 0))
    return pl.pallas_call(
        _tc_fin01_body,
        grid=(grid,),
        in_specs=[blk, blk, vec, vec, vec, vec, mat, mat],
        out_specs=[
            blk,
            pl.BlockSpec((2, BN, DH), lambda i: (0, i, 0)),
            blk,
        ],
        out_shape=[
            jax.ShapeDtypeStruct((N, D), _f32),
            jax.ShapeDtypeStruct((2, N, DH), _f32),
            jax.ShapeDtypeStruct((N, D), _f32),
        ],
    )(pre, h, s1, s2, g, be, wln, wrn)


def _tc_fin2_body(pre_ref, s1_ref, s2_ref, g_ref, be_ref,
                  w1a_ref, w1b_ref, a_ref, b_ref):
    h3 = _bn_relu(pre_ref[...], s1_ref[...], s2_ref[...],
                  g_ref[...], be_ref[...])
    a_ref[...] = _dot_t(h3, w1a_ref[...])
    b_ref[...] = _dot_t(h3, w1b_ref[...])


def _tc_fin2(pre, s1, s2, g, be, w1a, w1b):
    grid = N // BN
    vec = pl.BlockSpec((1, D), lambda i: (0, 0))
    return pl.pallas_call(
        _tc_fin2_body,
        grid=(grid,),
        in_specs=[
            pl.BlockSpec((BN, D), lambda i: (i, 0)),
            vec, vec, vec, vec,
            pl.BlockSpec((H, D), lambda i: (0, 0)),
            pl.BlockSpec((H, D), lambda i: (0, 0)),
        ],
        out_specs=[
            pl.BlockSpec((BN, H), lambda i: (i, 0)),
            pl.BlockSpec((BN, H), lambda i: (i, 0)),
        ],
        out_shape=[
            jax.ShapeDtypeStruct((N, H), _f32),
            jax.ShapeDtypeStruct((N, H), _f32),
        ],
    )(pre, s1, s2, g, be, w1a, w1b)


def _tc_edge_body(c_ref, b1_ref, w2_ref, b2_ref, w3_ref, b3_ref, out_ref):
    z = jnp.maximum(c_ref[...] + b1_ref[...], 0.0)
    z2 = jnp.maximum(_dot_t(z, w2_ref[...]) + b2_ref[...], 0.0)
    out_ref[...] = jnp.sum(z2 * w3_ref[...], axis=1, keepdims=True) + b3_ref[...]


def _tc_edge(cc, b1, w2, b2, w3, b3):
    grid = PE // BE
    return pl.pallas_call(
        _tc_edge_body,
        grid=(grid,),
        in_specs=[
            pl.BlockSpec((BE, DH), lambda i: (i, 0)),
            pl.BlockSpec((1, H), lambda i: (0, 0)),
            pl.BlockSpec((H // 2, H), lambda i: (0, 0)),
            pl.BlockSpec((1, H // 2), lambda i: (0, 0)),
            pl.BlockSpec((1, H // 2), lambda i: (0, 0)),
            pl.BlockSpec((1, 1), lambda i: (0, 0)),
        ],
        out_specs=pl.BlockSpec((BE, 1), lambda i: (i, 0)),
        out_shape=jax.ShapeDtypeStruct((PE, 1), _f32),
    )(cc, b1, w2, b2, w3, b3)


def kernel(x, edge_index, pred_edges, Wl0, bl0, Wr0, g0, be0, Wl1, bl1, Wr1,
           g1, be1, Wl2, bl2, Wr2, g2, be2, W1, b1, W2, b2, W3, b3):
    pad = EPAD - E
    srcp = jnp.concatenate(
        [edge_index[0], jnp.zeros((pad,), _i32)]).reshape(ROWS, 128)
    dstp = jnp.concatenate(
        [edge_index[1], jnp.full((pad,), PAD_DST, _i32)]).reshape(ROWS, 128)
    psp = jnp.concatenate(
        [pred_edges[0], jnp.zeros((pad,), _i32)]).reshape(ROWS, 128)
    ptp = jnp.concatenate(
        [pred_edges[1], jnp.zeros((pad,), _i32)]).reshape(ROWS, 128)

    cnt2 = _sc_count(dstp).reshape(2, NACC, 128)

    r1d = lambda v: v.reshape(1, -1)

    m01, r = _tc_i0(x, Wl0, Wr0)
    agg2 = _sc_agg(m01.reshape(2 * N, DH), srcp, dstp).reshape(2, NACC, DH)
    pre, s1, s2 = _tc_stats(agg2, cnt2, r, r1d(bl0))
    h, m01, r = _tc_fin01(pre, x, s1, s2, r1d(g0), r1d(be0), Wl1, Wr1)

    agg2 = _sc_agg(m01.reshape(2 * N, DH), srcp, dstp).reshape(2, NACC, DH)
    pre, s1, s2 = _tc_stats(agg2, cnt2, r, r1d(bl1))
    h, m01, r = _tc_fin01(pre, h, s1, s2, r1d(g1), r1d(be1), Wl2, Wr2)

    agg2 = _sc_agg(m01.reshape(2 * N, DH), srcp, dstp).reshape(2, NACC, DH)
    pre, s1, s2 = _tc_stats(agg2, cnt2, r, r1d(bl2))
    a_nodes, b_nodes = _tc_fin2(pre, s1, s2, r1d(g2), r1d(be2),
                                W1[:, :D], W1[:, D:])

    cc = _sc_pred(a_nodes, b_nodes, psp, ptp)
    z = _tc_edge(cc[:PE], r1d(b1), W2, r1d(b2), W3, b3.reshape(1, 1))
    return z.reshape(-1)
